# baseline (speedup 1.0000x reference)
import functools

import jax
import jax.numpy as jnp
from jax import lax
from jax.experimental import pallas as pl
from jax.experimental.pallas import tpu as pltpu
from jax.experimental.pallas import tpu_sc as plsc

H = 128
HEADS = 4
CPH = 32
NL = 4
OUT = 10

NW = 32
NPW = 320
NPAD = NW * NPW
CHUNK = 128
NEG = -1e30


_BLK = 1024


def _ln(x, g, b):
    m = jnp.mean(x, axis=-1, keepdims=True)
    v = jnp.var(x, axis=-1, keepdims=True)
    return (x - m) / jnp.sqrt(v + 1e-5) * g + b


def _gelu(x):
    return 0.5 * x * (1.0 + lax.erf(x * 0.7071067811865476))


def _r2(a):
    return a.reshape(1, -1)


def _pre_stage(x, p):
    n = x.shape[0]
    xp = jnp.pad(x, ((0, NPAD - n), (0, 0)))

    def body(x_ref, spW1, spb1, spg1, spbe1, spW2, spb2, spg2, spbe2,
             feW1, feb1, feg1, febe1, feW2, feb2, feg2, febe2,
             fuW1, fub1, fug1, fube1, fuW2, fub2, fug2, fube2,
             Wl, bl, Wr, br,
             h_ref, xl_ref, xr_ref):
        xv = x_ref[...]
        s = _gelu(_ln(jnp.dot(xv, spW1[...],
                              preferred_element_type=jnp.float32) + spb1[...],
                      spg1[...], spbe1[...]))
        s = _gelu(_ln(jnp.dot(s, spW2[...],
                              preferred_element_type=jnp.float32) + spb2[...],
                      spg2[...], spbe2[...]))
        f = _gelu(_ln(jnp.dot(xv, feW1[...],
                              preferred_element_type=jnp.float32) + feb1[...],
                      feg1[...], febe1[...]))
        f = _ln(jnp.dot(f, feW2[...],
                        preferred_element_type=jnp.float32) + feb2[...],
                feg2[...], febe2[...])
        h = jnp.concatenate([s, f], axis=1)
        h = _gelu(_ln(jnp.dot(h, fuW1[...],
                              preferred_element_type=jnp.float32) + fub1[...],
                      fug1[...], fube1[...]))
        h = _gelu(_ln(jnp.dot(h, fuW2[...],
                              preferred_element_type=jnp.float32) + fub2[...],
                      fug2[...], fube2[...]))
        h_ref[...] = h
        xl_ref[...] = jnp.dot(h, Wl[...],
                              preferred_element_type=jnp.float32) + bl[...]
        xr_ref[...] = jnp.dot(h, Wr[...],
                              preferred_element_type=jnp.float32) + br[...]

    g0 = p['gat0']
    spW1f = jnp.zeros((H, H), jnp.float32).at[:3, :].set(p['sp_W1'])
    feW1f = jnp.zeros((H, H), jnp.float32).at[3:, :].set(p['fe_W1'])
    ws = [spW1f, _r2(p['sp_b1']), _r2(p['sp_g1']), _r2(p['sp_be1']),
          p['sp_W2'], _r2(p['sp_b2']), _r2(p['sp_g2']), _r2(p['sp_be2']),
          feW1f, _r2(p['fe_b1']), _r2(p['fe_g1']), _r2(p['fe_be1']),
          p['fe_W2'], _r2(p['fe_b2']), _r2(p['fe_g2']), _r2(p['fe_be2']),
          p['fu_W1'], _r2(p['fu_b1']), _r2(p['fu_g1']), _r2(p['fu_be1']),
          p['fu_W2'], _r2(p['fu_b2']), _r2(p['fu_g2']), _r2(p['fu_be2']),
          g0['Wl'], _r2(g0['bl']), g0['Wr'], _r2(g0['br'])]
    w_specs = [pl.BlockSpec(w.shape, lambda i: (0, 0)) for w in ws]
    out = pl.pallas_call(
        body,
        grid=(NPAD // _BLK,),
        in_specs=[pl.BlockSpec((_BLK, H), lambda i: (i, 0))] + w_specs,
        out_specs=[pl.BlockSpec((_BLK, H), lambda i: (i, 0))] * 3,
        out_shape=[jax.ShapeDtypeStruct((NPAD, H), jnp.float32)] * 3,
    )(xp, *ws)
    return out


def _mid_stage(i, gat_out, h_in, hf_in, p):
    g = p['gat%d' % i]
    last = (i == NL - 1)
    lw = jax.nn.softmax(p['layer_weights'])
    wi = lw[i].reshape(1, 1)

    if not last:
        gn = p['gat%d' % (i + 1)]
        ws = [g['bias'].reshape(1, H), p['ln%d_g' % i].reshape(1, H),
              p['ln%d_b' % i].reshape(1, H),
              p['ffn%d_W1' % i], p['ffn%d_b1' % i].reshape(1, 2 * H),
              p['ffn%d_W2' % i], p['ffn%d_b2' % i].reshape(1, H),
              wi, gn['Wl'], gn['bl'].reshape(1, H), gn['Wr'],
              gn['br'].reshape(1, H)]

        def body(gat_ref, h_ref, hf_ref, bias, lng, lnb, W1, b1, W2, b2,
                 wref, Wl, bl, Wr, br, hout, hfout, xlout, xrout):
            hv = gat_ref[...] + bias[...]
            hv = _gelu(_ln(hv, lng[...], lnb[...]))
            h1 = hv + h_ref[...]
            ff = _gelu(jnp.dot(h1, W1[...],
                               preferred_element_type=jnp.float32) + b1[...])
            ff = jnp.dot(ff, W2[...],
                         preferred_element_type=jnp.float32) + b2[...]
            h2 = h1 + ff
            hout[...] = h2
            hfout[...] = hf_ref[...] + wref[0, 0] * h2
            xlout[...] = jnp.dot(h2, Wl[...],
                                 preferred_element_type=jnp.float32) + bl[...]
            xrout[...] = jnp.dot(h2, Wr[...],
                                 preferred_element_type=jnp.float32) + br[...]

        n_out = 4
    else:
        W3p = jnp.pad(p['cl_W3'], ((0, 0), (0, H - OUT)))
        b3p = jnp.pad(p['cl_b3'], (0, H - OUT)).reshape(1, H)
        ws = [g['bias'].reshape(1, H), p['ln%d_g' % i].reshape(1, H),
              p['ln%d_b' % i].reshape(1, H),
              p['ffn%d_W1' % i], p['ffn%d_b1' % i].reshape(1, 2 * H),
              p['ffn%d_W2' % i], p['ffn%d_b2' % i].reshape(1, H),
              wi,
              p['cl_W1'], p['cl_b1'].reshape(1, 2 * H),
              p['cl_g1'].reshape(1, 2 * H), p['cl_be1'].reshape(1, 2 * H),
              p['cl_W2'], p['cl_b2'].reshape(1, H),
              p['cl_g2'].reshape(1, H), p['cl_be2'].reshape(1, H),
              W3p, b3p]

        def body(gat_ref, h_ref, hf_ref, bias, lng, lnb, W1, b1, W2, b2,
                 wref, cW1, cb1, cg1, cbe1, cW2, cb2, cg2, cbe2, cW3, cb3,
                 oref):
            hv = gat_ref[...] + bias[...]
            hv = _gelu(_ln(hv, lng[...], lnb[...]))
            h1 = hv + h_ref[...]
            ff = _gelu(jnp.dot(h1, W1[...],
                               preferred_element_type=jnp.float32) + b1[...])
            ff = jnp.dot(ff, W2[...],
                         preferred_element_type=jnp.float32) + b2[...]
            h2 = h1 + ff
            hf = hf_ref[...] + wref[0, 0] * h2
            o = _gelu(_ln(jnp.dot(hf, cW1[...],
                                  preferred_element_type=jnp.float32) + cb1[...],
                          cg1[...], cbe1[...]))
            o = _gelu(_ln(jnp.dot(o, cW2[...],
                                  preferred_element_type=jnp.float32) + cb2[...],
                          cg2[...], cbe2[...]))
            oref[...] = jnp.dot(o, cW3[...],
                                preferred_element_type=jnp.float32) + cb3[...]

        n_out = 1

    w_specs = [pl.BlockSpec(w.shape, lambda i: (0, 0)) for w in ws]
    outs = pl.pallas_call(
        body,
        grid=(NPAD // _BLK,),
        in_specs=[pl.BlockSpec((_BLK, H), lambda i: (i, 0))] * 3 + w_specs,
        out_specs=[pl.BlockSpec((_BLK, H), lambda i: (i, 0))] * n_out,
        out_shape=[jax.ShapeDtypeStruct((NPAD, H), jnp.float32)] * n_out,
    )(gat_out, h_in, hf_in, *ws)
    return outs if not last else outs


_GD = lax.GatherDimensionNumbers(
    offset_dims=(), collapsed_slice_dims=(0,), start_index_map=(0,))


def _perm(v, idx):
    return lax.gather(v, idx[:, None], _GD, (1,),
                      mode=lax.GatherScatterMode.PROMISE_IN_BOUNDS)


def _bcast(v, lane):
    return _perm(v, jnp.full((16,), lane, jnp.int32))


@functools.lru_cache(maxsize=1)
def _gat_sc_kernel():
    mesh = plsc.VectorSubcoreMesh(core_axis_name="c", subcore_axis_name="s")

    @functools.partial(
        pl.kernel,
        mesh=mesh,
        out_type=jax.ShapeDtypeStruct((NPAD, H), jnp.float32),
        scratch_types=[
            pltpu.VMEM((CHUNK,), jnp.int32),
            pltpu.VMEM((CHUNK,), jnp.int32),
            pltpu.VMEM((CHUNK, H), jnp.float32),
            pltpu.VMEM((CHUNK, H), jnp.float32),
            pltpu.VMEM_SHARED((16, CHUNK), jnp.int32),
            pltpu.VMEM_SHARED((16, CHUNK), jnp.float32),
            pltpu.VMEM_SHARED((16, 128), jnp.int32),
            pltpu.SMEM((CHUNK,), jnp.int32),
            pltpu.SMEM((CHUNK,), jnp.int32),
            pltpu.SMEM((CHUNK,), jnp.float32),
            pltpu.SMEM((CHUNK,), jnp.float32),
            pltpu.SMEM((128,), jnp.int32),
            pltpu.VMEM((128,), jnp.float32),
            pltpu.VMEM((128,), jnp.float32),
            pltpu.VMEM((NPW + 1, H), jnp.float32),
            pltpu.VMEM((NPW + 1, H), jnp.float32),
            pltpu.VMEM(((NPW + 1) * 16,), jnp.float32),
            pltpu.VMEM((CHUNK * 16,), jnp.float32),
            pltpu.SemaphoreType.DMA,
            pltpu.SemaphoreType.DMA,
        ],
    )
    def k(xl_hbm, xr_hbm, bsrc_hbm, bdst_hbm, bw_hbm, offs_hbm,
          we_hbm, att_hbm, out_hbm,
          srcv0, srcv1, xlr0, xlr1, dstg, wg, offg,
          dsts0, dsts1, ws0, ws1, offs, web, attb,
          xrown, acc, state, scbuf, sem0, sem1):
        cid = lax.axis_index("c")
        sid = lax.axis_index("s")
        wid = sid * 2 + cid
        base = wid * NPW

        srcv = (srcv0, srcv1)
        xlr = (xlr0, xlr1)
        dsts = (dsts0, dsts1)
        wss = (ws0, ws1)
        sems = (sem0, sem1)

        pltpu.sync_copy(offs_hbm.at[0], offg.at[sid])
        pltpu.sync_copy(offg.at[sid], offs)
        e_row = offs[wid]
        cnt = offs[NW + wid]
        pltpu.sync_copy(we_hbm.at[0], web)
        pltpu.sync_copy(att_hbm.at[0], attb)
        pltpu.sync_copy(xr_hbm.at[pl.ds(base, NPW)], xrown.at[pl.ds(0, NPW)])

        def init_row(i, _):
            def init_g(g, _):
                acc[i, pl.ds(g * 16, 16)] = jnp.zeros((16,), jnp.float32)
                return 0
            lax.fori_loop(0, 8, init_g, 0, unroll=True)
            state[pl.ds(i * 16, 16)] = jnp.full((16,), NEG, jnp.float32)
            return 0
        lax.fori_loop(0, NPW + 1, init_row, 0)

        wes = tuple(web[pl.ds(g * 16, 16)] for g in range(8))
        ats = tuple(attb[pl.ds(g * 16, 16)] for g in range(8))
        lanes = lax.iota(jnp.int32, 16)
        r4 = lanes % 4
        rot8 = (lanes + 8) % 16
        rot4 = (lanes + 4) % 16
        rot2 = (lanes + 2) % 16
        rot1 = (lanes + 1) % 16
        rm1 = (lanes + 15) % 16

        cnt4 = ((cnt + 3) // 4) * 4
        nch = (cnt4 + CHUNK - 1) // CHUNK

        def issue(c, b):
            off = pl.multiple_of((e_row + c) * CHUNK, 8)
            row = e_row + c
            pltpu.sync_copy(bsrc_hbm.at[pl.ds(off, CHUNK)], srcv[b])
            pltpu.sync_copy(bdst_hbm.at[row], dstg.at[sid])
            pltpu.sync_copy(dstg.at[sid], dsts[b])
            pltpu.sync_copy(bw_hbm.at[row], wg.at[sid])
            pltpu.sync_copy(wg.at[sid], wss[b])
            pltpu.async_copy(xl_hbm.at[srcv[b]], xlr[b], sems[b])

        def score_edge(e, dref, wref, xref):
            d = dref[e] - base
            w = wref[e]
            xl = tuple(xref[e, pl.ds(g * 16, 16)] for g in range(8))
            xr = tuple(xrown[d, pl.ds(g * 16, 16)] for g in range(8))
            ps = []
            for g in range(8):
                t = xl[g] + xr[g] + wes[g] * w
                t = jnp.maximum(t, 0.2 * t)
                ps.append(t * ats[g])
            hv = [ps[0] + ps[1], ps[2] + ps[3], ps[4] + ps[5], ps[6] + ps[7]]
            hv = [v + _perm(v, rot8) for v in hv]
            hv = [v + _perm(v, rot4) for v in hv]
            wv = jnp.where(lanes < 8,
                           jnp.where(lanes < 4, hv[0], hv[1]),
                           jnp.where(lanes < 12, hv[2], hv[3]))
            wv = wv + _perm(wv, rot2)
            sc = wv + _perm(wv, rot1)
            scbuf[pl.ds(e * 16, 16)] = sc

        def update_edge(e, dref, xref):
            d = dref[e] - base
            sc = scbuf[pl.ds(e * 16, 16)]
            xl = tuple(xref[e, pl.ds(g * 16, 16)] for g in range(8))
            st = state[pl.ds(d * 16, 16)]
            m_new = jnp.maximum(st, sc)
            f = jnp.exp(st - m_new)
            pv = jnp.exp(sc - m_new)
            fs = _perm(f, rm1)
            pvs = _perm(pv, rm1)
            dn_new = st * fs + pvs
            st_new = jnp.where(r4 < 1, m_new,
                               jnp.where(r4 < 2, dn_new, st))
            state[pl.ds(d * 16, 16)] = st_new

            for h in range(4):
                fh = _bcast(f, 4 * h)
                ph = _bcast(pv, 4 * h)
                for g in (2 * h, 2 * h + 1):
                    a = acc[d, pl.ds(g * 16, 16)]
                    acc[d, pl.ds(g * 16, 16)] = a * fh + ph * xl[g]

        def process(c, b):
            ecnt = jnp.minimum(cnt4 - c * CHUNK, CHUNK)

            def sgrp(gi, _):
                for j in range(4):
                    score_edge(gi * 4 + j, dsts[b], wss[b], xlr[b])
                return 0
            lax.fori_loop(0, ecnt // 4, sgrp, 0)

            def ugrp(gi, _):
                for j in range(4):
                    update_edge(gi * 4 + j, dsts[b], xlr[b])
                return 0
            lax.fori_loop(0, ecnt // 4, ugrp, 0)

        @pl.when(nch > 0)
        def _():
            issue(0, 0)

        def pair_body(cp, _):
            for b in (0, 1):
                c = cp * 2 + b

                @pl.when(c < nch)
                def _():
                    pltpu.make_async_copy(
                        xl_hbm.at[srcv[b]], xlr[b], sems[b]).wait()

                    @pl.when(c + 1 < nch)
                    def _():
                        issue(c + 1, 1 - b)
                    process(c, b)
            return 0
        lax.fori_loop(0, (nch + 1) // 2, pair_body, 0)

        def flush_body(i, _):
            st = state[pl.ds(i * 16, 16)]
            dnv = _perm(st, rot1)
            rdn = 1.0 / (dnv + 1e-16)
            for h in range(4):
                rh = _bcast(rdn, 4 * h)
                for g in (2 * h, 2 * h + 1):
                    acc[i, pl.ds(g * 16, 16)] = acc[i, pl.ds(g * 16, 16)] * rh
            return 0
        lax.fori_loop(0, NPW, flush_body, 0)
        pltpu.sync_copy(acc.at[pl.ds(0, NPW)], out_hbm.at[pl.ds(base, NPW)])

    return k


def _bucket_edges_jnp(src, dst, w):
    e = src.shape[0]
    b = dst // NPW
    order = jnp.argsort(b)
    bs = b[order]
    starts = jnp.searchsorted(bs, jnp.arange(NW + 1, dtype=jnp.int32),
                              side='left').astype(jnp.int32)
    counts = starts[1:] - starts[:-1]
    caps = ((counts + CHUNK - 1) // CHUNK) * CHUNK
    base = jnp.concatenate([jnp.zeros((1,), jnp.int32),
                            jnp.cumsum(caps, dtype=jnp.int32)])[:NW]
    j = jnp.arange(e, dtype=jnp.int32)
    pos = base[bs] + (j - starts[bs])
    ep = e + NW * CHUNK
    src_s = src[order]
    dst_s = dst[order]
    w_s = w[order]
    rb = jnp.repeat(jnp.arange(NW, dtype=jnp.int32), caps,
                    total_repeat_length=ep)
    slotoff = jnp.arange(ep, dtype=jnp.int32) - base[rb]
    valid = slotoff < counts[rb]
    gidx = jnp.minimum(starts[rb] + slotoff, e - 1)
    bsrc = jnp.where(valid, src_s[gidx], 0)
    bdst = jnp.where(valid, dst_s[gidx], (rb + 1) * NPW)
    bw = jnp.where(valid, w_s[gidx], jnp.float32(0))
    offs = jnp.zeros((1, 128), jnp.int32)
    offs = offs.at[0, :NW].set(base // CHUNK)
    offs = offs.at[0, NW:2 * NW].set(counts)
    return bsrc, bdst.reshape(-1, CHUNK), bw.reshape(-1, CHUNK), offs


def kernel(x, edge_index, edge_weight, params):
    p = params
    n = x.shape[0]
    src = edge_index[0]
    dst = edge_index[1]

    bsrc, bdst, bw, offs = _bucket_edges_jnp(src, dst, edge_weight)

    h, xl, xr = _pre_stage(x, p)
    gat = _gat_sc_kernel()
    hf = jnp.zeros((NPAD, H), jnp.float32)
    for i in range(NL):
        g = p['gat%d' % i]
        we = g['We'].reshape(1, H)
        att = g['att'].reshape(1, H)
        gout = gat(xl, xr, bsrc, bdst, bw, offs, we, att)
        outs = _mid_stage(i, gout, h, hf, p)
        if i < NL - 1:
            h, hf, xl, xr = outs
        else:
            o = outs[0]
    return o[:n, :OUT]


# speedup vs baseline: 1.1649x; 1.1649x over previous
import functools

import jax
import jax.numpy as jnp
from jax import lax
from jax.experimental import pallas as pl
from jax.experimental.pallas import tpu as pltpu
from jax.experimental.pallas import tpu_sc as plsc

H = 128
HEADS = 4
CPH = 32
NL = 4
OUT = 10

NW = 32
NPW = 320
NPAD = NW * NPW
CHUNK = 128
NEG = -1e30


_BLK = 1024


def _ln(x, g, b):
    m = jnp.mean(x, axis=-1, keepdims=True)
    v = jnp.var(x, axis=-1, keepdims=True)
    return (x - m) / jnp.sqrt(v + 1e-5) * g + b


def _gelu(x):
    return 0.5 * x * (1.0 + lax.erf(x * 0.7071067811865476))


def _r2(a):
    return a.reshape(1, -1)


def _pre_stage(x, p):
    n = x.shape[0]
    xp = jnp.pad(x, ((0, NPAD - n), (0, 0)))

    def body(x_ref, spW1, spb1, spg1, spbe1, spW2, spb2, spg2, spbe2,
             feW1, feb1, feg1, febe1, feW2, feb2, feg2, febe2,
             fuW1, fub1, fug1, fube1, fuW2, fub2, fug2, fube2,
             Wl, bl, Wr, br,
             h_ref, xl_ref, xr_ref):
        xv = x_ref[...]
        s = _gelu(_ln(jnp.dot(xv, spW1[...],
                              preferred_element_type=jnp.float32) + spb1[...],
                      spg1[...], spbe1[...]))
        s = _gelu(_ln(jnp.dot(s, spW2[...],
                              preferred_element_type=jnp.float32) + spb2[...],
                      spg2[...], spbe2[...]))
        f = _gelu(_ln(jnp.dot(xv, feW1[...],
                              preferred_element_type=jnp.float32) + feb1[...],
                      feg1[...], febe1[...]))
        f = _ln(jnp.dot(f, feW2[...],
                        preferred_element_type=jnp.float32) + feb2[...],
                feg2[...], febe2[...])
        h = jnp.concatenate([s, f], axis=1)
        h = _gelu(_ln(jnp.dot(h, fuW1[...],
                              preferred_element_type=jnp.float32) + fub1[...],
                      fug1[...], fube1[...]))
        h = _gelu(_ln(jnp.dot(h, fuW2[...],
                              preferred_element_type=jnp.float32) + fub2[...],
                      fug2[...], fube2[...]))
        h_ref[...] = h
        xl_ref[...] = jnp.dot(h, Wl[...],
                              preferred_element_type=jnp.float32) + bl[...]
        xr_ref[...] = jnp.dot(h, Wr[...],
                              preferred_element_type=jnp.float32) + br[...]

    g0 = p['gat0']
    spW1f = jnp.zeros((H, H), jnp.float32).at[:3, :].set(p['sp_W1'])
    feW1f = jnp.zeros((H, H), jnp.float32).at[3:, :].set(p['fe_W1'])
    ws = [spW1f, _r2(p['sp_b1']), _r2(p['sp_g1']), _r2(p['sp_be1']),
          p['sp_W2'], _r2(p['sp_b2']), _r2(p['sp_g2']), _r2(p['sp_be2']),
          feW1f, _r2(p['fe_b1']), _r2(p['fe_g1']), _r2(p['fe_be1']),
          p['fe_W2'], _r2(p['fe_b2']), _r2(p['fe_g2']), _r2(p['fe_be2']),
          p['fu_W1'], _r2(p['fu_b1']), _r2(p['fu_g1']), _r2(p['fu_be1']),
          p['fu_W2'], _r2(p['fu_b2']), _r2(p['fu_g2']), _r2(p['fu_be2']),
          g0['Wl'], _r2(g0['bl']), g0['Wr'], _r2(g0['br'])]
    w_specs = [pl.BlockSpec(w.shape, lambda i: (0, 0)) for w in ws]
    out = pl.pallas_call(
        body,
        grid=(NPAD // _BLK,),
        in_specs=[pl.BlockSpec((_BLK, H), lambda i: (i, 0))] + w_specs,
        out_specs=[pl.BlockSpec((_BLK, H), lambda i: (i, 0))] * 3,
        out_shape=[jax.ShapeDtypeStruct((NPAD, H), jnp.float32)] * 3,
    )(xp, *ws)
    return out


def _mid_stage(i, gat_out, h_in, hf_in, p):
    g = p['gat%d' % i]
    last = (i == NL - 1)
    lw = jax.nn.softmax(p['layer_weights'])
    wi = lw[i].reshape(1, 1)

    if not last:
        gn = p['gat%d' % (i + 1)]
        ws = [g['bias'].reshape(1, H), p['ln%d_g' % i].reshape(1, H),
              p['ln%d_b' % i].reshape(1, H),
              p['ffn%d_W1' % i], p['ffn%d_b1' % i].reshape(1, 2 * H),
              p['ffn%d_W2' % i], p['ffn%d_b2' % i].reshape(1, H),
              wi, gn['Wl'], gn['bl'].reshape(1, H), gn['Wr'],
              gn['br'].reshape(1, H)]

        def body(gat_ref, h_ref, hf_ref, bias, lng, lnb, W1, b1, W2, b2,
                 wref, Wl, bl, Wr, br, hout, hfout, xlout, xrout):
            hv = gat_ref[...] + bias[...]
            hv = _gelu(_ln(hv, lng[...], lnb[...]))
            h1 = hv + h_ref[...]
            ff = _gelu(jnp.dot(h1, W1[...],
                               preferred_element_type=jnp.float32) + b1[...])
            ff = jnp.dot(ff, W2[...],
                         preferred_element_type=jnp.float32) + b2[...]
            h2 = h1 + ff
            hout[...] = h2
            hfout[...] = hf_ref[...] + wref[0, 0] * h2
            xlout[...] = jnp.dot(h2, Wl[...],
                                 preferred_element_type=jnp.float32) + bl[...]
            xrout[...] = jnp.dot(h2, Wr[...],
                                 preferred_element_type=jnp.float32) + br[...]

        n_out = 4
    else:
        W3p = jnp.pad(p['cl_W3'], ((0, 0), (0, H - OUT)))
        b3p = jnp.pad(p['cl_b3'], (0, H - OUT)).reshape(1, H)
        ws = [g['bias'].reshape(1, H), p['ln%d_g' % i].reshape(1, H),
              p['ln%d_b' % i].reshape(1, H),
              p['ffn%d_W1' % i], p['ffn%d_b1' % i].reshape(1, 2 * H),
              p['ffn%d_W2' % i], p['ffn%d_b2' % i].reshape(1, H),
              wi,
              p['cl_W1'], p['cl_b1'].reshape(1, 2 * H),
              p['cl_g1'].reshape(1, 2 * H), p['cl_be1'].reshape(1, 2 * H),
              p['cl_W2'], p['cl_b2'].reshape(1, H),
              p['cl_g2'].reshape(1, H), p['cl_be2'].reshape(1, H),
              W3p, b3p]

        def body(gat_ref, h_ref, hf_ref, bias, lng, lnb, W1, b1, W2, b2,
                 wref, cW1, cb1, cg1, cbe1, cW2, cb2, cg2, cbe2, cW3, cb3,
                 oref):
            hv = gat_ref[...] + bias[...]
            hv = _gelu(_ln(hv, lng[...], lnb[...]))
            h1 = hv + h_ref[...]
            ff = _gelu(jnp.dot(h1, W1[...],
                               preferred_element_type=jnp.float32) + b1[...])
            ff = jnp.dot(ff, W2[...],
                         preferred_element_type=jnp.float32) + b2[...]
            h2 = h1 + ff
            hf = hf_ref[...] + wref[0, 0] * h2
            o = _gelu(_ln(jnp.dot(hf, cW1[...],
                                  preferred_element_type=jnp.float32) + cb1[...],
                          cg1[...], cbe1[...]))
            o = _gelu(_ln(jnp.dot(o, cW2[...],
                                  preferred_element_type=jnp.float32) + cb2[...],
                          cg2[...], cbe2[...]))
            oref[...] = jnp.dot(o, cW3[...],
                                preferred_element_type=jnp.float32) + cb3[...]

        n_out = 1

    w_specs = [pl.BlockSpec(w.shape, lambda i: (0, 0)) for w in ws]
    outs = pl.pallas_call(
        body,
        grid=(NPAD // _BLK,),
        in_specs=[pl.BlockSpec((_BLK, H), lambda i: (i, 0))] * 3 + w_specs,
        out_specs=[pl.BlockSpec((_BLK, H), lambda i: (i, 0))] * n_out,
        out_shape=[jax.ShapeDtypeStruct((NPAD, H), jnp.float32)] * n_out,
    )(gat_out, h_in, hf_in, *ws)
    return outs if not last else outs


_GD = lax.GatherDimensionNumbers(
    offset_dims=(), collapsed_slice_dims=(0,), start_index_map=(0,))


def _perm(v, idx):
    return lax.gather(v, idx[:, None], _GD, (1,),
                      mode=lax.GatherScatterMode.PROMISE_IN_BOUNDS)


def _bcast(v, lane):
    return _perm(v, jnp.full((16,), lane, jnp.int32))


@functools.lru_cache(maxsize=1)
def _gat_sc_kernel():
    mesh = plsc.VectorSubcoreMesh(core_axis_name="c", subcore_axis_name="s")

    @functools.partial(
        pl.kernel,
        mesh=mesh,
        out_type=jax.ShapeDtypeStruct((NPAD, H), jnp.float32),
        scratch_types=[
            pltpu.VMEM((CHUNK,), jnp.int32),
            pltpu.VMEM((CHUNK,), jnp.int32),
            pltpu.VMEM((CHUNK, H), jnp.float32),
            pltpu.VMEM((CHUNK, H), jnp.float32),
            pltpu.VMEM_SHARED((16, CHUNK), jnp.int32),
            pltpu.VMEM_SHARED((16, CHUNK), jnp.int32),
            pltpu.VMEM_SHARED((16, CHUNK), jnp.float32),
            pltpu.VMEM_SHARED((16, CHUNK), jnp.float32),
            pltpu.VMEM_SHARED((16, 128), jnp.int32),
            pltpu.SMEM((CHUNK,), jnp.int32),
            pltpu.SMEM((CHUNK,), jnp.int32),
            pltpu.SMEM((CHUNK,), jnp.float32),
            pltpu.SMEM((CHUNK,), jnp.float32),
            pltpu.SMEM((128,), jnp.int32),
            pltpu.VMEM((128,), jnp.float32),
            pltpu.VMEM((128,), jnp.float32),
            pltpu.VMEM((NPW + 1, H), jnp.float32),
            pltpu.VMEM((NPW + 1, H), jnp.float32),
            pltpu.VMEM(((NPW + 1) * 16,), jnp.float32),
            pltpu.SemaphoreType.DMA,
            pltpu.SemaphoreType.DMA,
        ],
    )
    def k(xl_hbm, xr_hbm, bsrc_hbm, bdst_hbm, bw_hbm, offs_hbm,
          we_hbm, att_hbm, out_hbm,
          srcv0, srcv1, xlr0, xlr1, dstg0, dstg1, wg0, wg1, offg,
          dsts0, dsts1, ws0, ws1, offs, web, attb,
          xrown, acc, state, sem0, sem1):
        cid = lax.axis_index("c")
        sid = lax.axis_index("s")
        wid = sid * 2 + cid
        base = wid * NPW

        srcv = (srcv0, srcv1)
        xlr = (xlr0, xlr1)
        dstg = (dstg0, dstg1)
        wg = (wg0, wg1)
        dsts = (dsts0, dsts1)
        wss = (ws0, ws1)
        sems = (sem0, sem1)

        pltpu.sync_copy(offs_hbm.at[0], dstg0.at[sid])
        pltpu.sync_copy(dstg0.at[sid], offs)
        e_row = offs[wid]
        cnt = offs[NW + wid]
        pltpu.sync_copy(we_hbm.at[0], web)
        pltpu.sync_copy(att_hbm.at[0], attb)
        pltpu.sync_copy(xr_hbm.at[pl.ds(base, NPW)], xrown.at[pl.ds(0, NPW)])

        def init_row(i, _):
            def init_g(g, _):
                acc[i, pl.ds(g * 16, 16)] = jnp.zeros((16,), jnp.float32)
                return 0
            lax.fori_loop(0, 8, init_g, 0, unroll=True)
            state[pl.ds(i * 16, 16)] = jnp.full((16,), NEG, jnp.float32)
            return 0
        lax.fori_loop(0, NPW + 1, init_row, 0)

        wes = tuple(web[pl.ds(g * 16, 16)] for g in range(8))
        ats = tuple(attb[pl.ds(g * 16, 16)] for g in range(8))
        lanes = lax.iota(jnp.int32, 16)
        r4 = lanes % 4
        rot8 = (lanes + 8) % 16
        rot4 = (lanes + 4) % 16
        rot2 = (lanes + 2) % 16
        rot1 = (lanes + 1) % 16
        rm1 = (lanes + 15) % 16

        cnt4 = ((cnt + 3) // 4) * 4
        nch = (cnt4 + CHUNK - 1) // CHUNK

        def issue_scalars(c, b):
            row = e_row + c
            pltpu.async_copy(bdst_hbm.at[row], dstg[b].at[sid], sems[b])
            pltpu.async_copy(bw_hbm.at[row], wg[b].at[sid], sems[b])

        def issue_gather(c, b):
            off = pl.multiple_of((e_row + c) * CHUNK, 8)
            pltpu.sync_copy(bsrc_hbm.at[pl.ds(off, CHUNK)], srcv[b])
            pltpu.async_copy(xl_hbm.at[srcv[b]], xlr[b], sems[b])

        def wait_chunk(c, b):
            row = e_row + c
            pltpu.make_async_copy(bdst_hbm.at[row], dstg[b].at[sid],
                                  sems[b]).wait()
            pltpu.make_async_copy(bw_hbm.at[row], wg[b].at[sid],
                                  sems[b]).wait()
            pltpu.make_async_copy(xl_hbm.at[srcv[b]], xlr[b], sems[b]).wait()
            pltpu.sync_copy(dstg[b].at[sid], dsts[b])
            pltpu.sync_copy(wg[b].at[sid], wss[b])

        def one_edge(e, dref, wref, xref):
            d = dref[e] - base
            w = wref[e]
            xl = tuple(xref[e, pl.ds(g * 16, 16)] for g in range(8))
            xr = tuple(xrown[d, pl.ds(g * 16, 16)] for g in range(8))
            ps = []
            for g in range(8):
                t = xl[g] + xr[g] + wes[g] * w
                t = jnp.maximum(t, 0.2 * t)
                ps.append(t * ats[g])
            hv = [ps[0] + ps[1], ps[2] + ps[3], ps[4] + ps[5], ps[6] + ps[7]]
            hv = [v + _perm(v, rot8) for v in hv]
            hv = [v + _perm(v, rot4) for v in hv]
            wv = jnp.where(lanes < 8,
                           jnp.where(lanes < 4, hv[0], hv[1]),
                           jnp.where(lanes < 12, hv[2], hv[3]))
            wv = wv + _perm(wv, rot2)
            sc = wv + _perm(wv, rot1)

            st = state[pl.ds(d * 16, 16)]
            m_new = jnp.maximum(st, sc)
            f = jnp.exp(st - m_new)
            pv = jnp.exp(sc - m_new)
            fs = _perm(f, rm1)
            pvs = _perm(pv, rm1)
            dn_new = st * fs + pvs
            st_new = jnp.where(r4 < 1, m_new,
                               jnp.where(r4 < 2, dn_new, st))
            state[pl.ds(d * 16, 16)] = st_new

            for h in range(4):
                fh = _bcast(f, 4 * h)
                ph = _bcast(pv, 4 * h)
                for g in (2 * h, 2 * h + 1):
                    a = acc[d, pl.ds(g * 16, 16)]
                    acc[d, pl.ds(g * 16, 16)] = a * fh + ph * xl[g]

        def process(c, b):
            ecnt = jnp.minimum(cnt4 - c * CHUNK, CHUNK)

            def grp(gi, _):
                for j in range(4):
                    one_edge(gi * 4 + j, dsts[b], wss[b], xlr[b])
                return 0
            lax.fori_loop(0, ecnt // 4, grp, 0)

        @pl.when(nch > 0)
        def _():
            issue_scalars(0, 0)
            issue_gather(0, 0)

        def pair_body(cp, _):
            for b in (0, 1):
                c = cp * 2 + b

                @pl.when(c < nch)
                def _():
                    @pl.when(c + 1 < nch)
                    def _():
                        issue_scalars(c + 1, 1 - b)
                    wait_chunk(c, b)

                    @pl.when(c + 1 < nch)
                    def _():
                        issue_gather(c + 1, 1 - b)
                    process(c, b)
            return 0
        lax.fori_loop(0, (nch + 1) // 2, pair_body, 0)

        def flush_body(i, _):
            st = state[pl.ds(i * 16, 16)]
            dnv = _perm(st, rot1)
            rdn = 1.0 / (dnv + 1e-16)
            for h in range(4):
                rh = _bcast(rdn, 4 * h)
                for g in (2 * h, 2 * h + 1):
                    acc[i, pl.ds(g * 16, 16)] = acc[i, pl.ds(g * 16, 16)] * rh
            return 0
        lax.fori_loop(0, NPW, flush_body, 0)
        pltpu.sync_copy(acc.at[pl.ds(0, NPW)], out_hbm.at[pl.ds(base, NPW)])

    return k


def _bucket_edges_jnp(src, dst, w):
    e = src.shape[0]
    b = dst // NPW
    order = jnp.argsort(b)
    bs = b[order]
    starts = jnp.searchsorted(bs, jnp.arange(NW + 1, dtype=jnp.int32),
                              side='left').astype(jnp.int32)
    counts = starts[1:] - starts[:-1]
    caps = ((counts + CHUNK - 1) // CHUNK) * CHUNK
    base = jnp.concatenate([jnp.zeros((1,), jnp.int32),
                            jnp.cumsum(caps, dtype=jnp.int32)])[:NW]
    j = jnp.arange(e, dtype=jnp.int32)
    pos = base[bs] + (j - starts[bs])
    ep = e + NW * CHUNK
    src_s = src[order]
    dst_s = dst[order]
    w_s = w[order]
    rb = jnp.repeat(jnp.arange(NW, dtype=jnp.int32), caps,
                    total_repeat_length=ep)
    slotoff = jnp.arange(ep, dtype=jnp.int32) - base[rb]
    valid = slotoff < counts[rb]
    gidx = jnp.minimum(starts[rb] + slotoff, e - 1)
    bsrc = jnp.where(valid, src_s[gidx], 0)
    bdst = jnp.where(valid, dst_s[gidx], (rb + 1) * NPW)
    bw = jnp.where(valid, w_s[gidx], jnp.float32(0))
    offs = jnp.zeros((1, 128), jnp.int32)
    offs = offs.at[0, :NW].set(base // CHUNK)
    offs = offs.at[0, NW:2 * NW].set(counts)
    return bsrc, bdst.reshape(-1, CHUNK), bw.reshape(-1, CHUNK), offs


def kernel(x, edge_index, edge_weight, params):
    p = params
    n = x.shape[0]
    src = edge_index[0]
    dst = edge_index[1]

    bsrc, bdst, bw, offs = _bucket_edges_jnp(src, dst, edge_weight)

    h, xl, xr = _pre_stage(x, p)
    gat = _gat_sc_kernel()
    hf = jnp.zeros((NPAD, H), jnp.float32)
    for i in range(NL):
        g = p['gat%d' % i]
        we = g['We'].reshape(1, H)
        att = g['att'].reshape(1, H)
        gout = gat(xl, xr, bsrc, bdst, bw, offs, we, att)
        outs = _mid_stage(i, gout, h, hf, p)
        if i < NL - 1:
            h, hf, xl, xr = outs
        else:
            o = outs[0]
    return o[:n, :OUT]
